# Initial kernel scaffold; baseline (speedup 1.0000x reference)
import jax
import jax.numpy as jnp
from jax.experimental import pallas as pl


def kernel(params, x, adj_indices, adj_values, adj_diff_indices, adj_diff_values):
    raise NotImplementedError("write your pallas kernel here")


# speedup vs baseline: 4.5921x; 4.5921x over previous
import functools

import jax
import jax.numpy as jnp
from jax.experimental import pallas as pl

EPS = 1e-5
K_NN = 10
_BN_INV = 1.0 / (1.0 + EPS) ** 0.5


def _elu(x):
    return jnp.where(x > 0, x, jnp.exp(jnp.minimum(x, 0.0)) - 1.0)


def _rows_block(n):
    for rb in (1000, 400, 200, 80, 40, 16, 8):
        if n % rb == 0:
            return rb
    return n


def _enc_kernel(x_ref, w1, b1, g1, be1, w2, b2, g2, be2, w3, b3, g3, be3,
                dw1, db1, dw2, db2, dw3, db3, gw1,
                fd_ref, fg1_ref, mse_ref):
    x = x_ref[...]
    z1 = _elu((jnp.dot(x, w1[...], preferred_element_type=jnp.float32)
               + b1[...]) * _BN_INV * g1[...] + be1[...])
    z2 = _elu((jnp.dot(z1, w2[...], preferred_element_type=jnp.float32)
               + b2[...]) * _BN_INV * g2[...] + be2[...])
    z3 = _elu((jnp.dot(z2, w3[...], preferred_element_type=jnp.float32)
               + b3[...]) * _BN_INV * g3[...] + be3[...])
    xd = _elu(jnp.dot(z3, dw1[...], preferred_element_type=jnp.float32) + db1[...])
    xd = _elu(jnp.dot(xd, dw2[...], preferred_element_type=jnp.float32) + db2[...])
    xd = _elu(jnp.dot(xd, dw3[...], preferred_element_type=jnp.float32) + db3[...])
    diff = xd - x
    part = jnp.sum(diff * diff)
    fd = jnp.concatenate([z1, z2, z3], axis=1)
    pad = fd_ref.shape[1] - fd.shape[1]
    if pad:
        fd = jnp.concatenate(
            [fd, jnp.zeros((fd.shape[0], pad), jnp.float32)], axis=1)
    fd_ref[...] = fd
    fg1_ref[...] = jnp.dot(fd, gw1[...], preferred_element_type=jnp.float32)

    @pl.when(pl.program_id(0) == 0)
    def _():
        mse_ref[...] = jnp.zeros_like(mse_ref)

    r = jax.lax.broadcasted_iota(jnp.int32, mse_ref.shape, 0)
    c = jax.lax.broadcasted_iota(jnp.int32, mse_ref.shape, 1)
    mse_ref[...] += jnp.where((r == 0) & (c == 0), part, 0.0)


def _run_encoder(p, x, fdim_pad):
    n, f = x.shape
    rb = _rows_block(n)
    grid = (n // rb,)
    h4 = p['enc_W1'].shape[1]
    h2 = p['enc_W2'].shape[1]
    h = p['enc_W3'].shape[1]
    g1w = p['g1']
    fg1_dim = g1w.shape[1]

    def row2(d):
        return pl.BlockSpec((1, d), lambda i: (0, 0))

    def full(a, b):
        return pl.BlockSpec((a, b), lambda i: (0, 0))

    in_specs = [
        pl.BlockSpec((rb, f), lambda i: (i, 0)),
        full(f, h4), row2(h4), row2(h4), row2(h4),
        full(h4, h2), row2(h2), row2(h2), row2(h2),
        full(h2, h), row2(h), row2(h), row2(h),
        full(h, h2), row2(h2),
        full(h2, h4), row2(h4),
        full(h4, f), row2(f),
        pl.BlockSpec((fdim_pad, fg1_dim), lambda i: (0, 0)),
    ]
    g1p = jnp.zeros((fdim_pad, fg1_dim), jnp.float32).at[:g1w.shape[0]].set(g1w)
    args = [x,
            p['enc_W1'], p['enc_b1'][None, :], p['bn1_g'][None, :], p['bn1_b'][None, :],
            p['enc_W2'], p['enc_b2'][None, :], p['bn2_g'][None, :], p['bn2_b'][None, :],
            p['enc_W3'], p['enc_b3'][None, :], p['bn3_g'][None, :], p['bn3_b'][None, :],
            p['dec_W1'], p['dec_b1'][None, :],
            p['dec_W2'], p['dec_b2'][None, :],
            p['dec_W3'], p['dec_b3'][None, :],
            g1p]
    fd, fg1, msep = pl.pallas_call(
        _enc_kernel,
        grid=grid,
        in_specs=in_specs,
        out_specs=[
            pl.BlockSpec((rb, fdim_pad), lambda i: (i, 0)),
            pl.BlockSpec((rb, fg1_dim), lambda i: (i, 0)),
            pl.BlockSpec((8, 128), lambda i: (0, 0)),
        ],
        out_shape=[
            jax.ShapeDtypeStruct((n, fdim_pad), jnp.float32),
            jax.ShapeDtypeStruct((n, fg1_dim), jnp.float32),
            jax.ShapeDtypeStruct((8, 128), jnp.float32),
        ],
    )(*args)
    mse = msep[0, 0] / (n * f)
    return fd, fg1, mse


def _topk_kernel(fb_ref, ftt_ref, idx_ref, *, n_valid, k_sel):
    fb = fb_ref[...]
    ftt = ftt_ref[...]
    mm = jnp.dot(fb, ftt, preferred_element_type=jnp.float32)
    sqr = jnp.sum(fb * fb, axis=1, keepdims=True)
    sqc = jnp.sum(ftt * ftt, axis=0, keepdims=True)
    d2 = sqr + sqc - 2.0 * mm
    col = jax.lax.broadcasted_iota(jnp.int32, d2.shape, 1)
    d2 = jnp.where(col < n_valid, d2, jnp.inf)
    idx_ref[...] = jnp.zeros_like(idx_ref)
    for k in range(k_sel):
        m = jnp.min(d2, axis=1, keepdims=True)
        sel = jnp.min(jnp.where(d2 == m, col, jnp.int32(2 ** 30)),
                      axis=1, keepdims=True)
        idx_ref[:, k:k + 1] = sel
        d2 = jnp.where(col == sel, jnp.inf, d2)


def _run_topk(fd_pad, n, k_sel):
    fp = fd_pad.shape[1]
    n_pad = ((n + 127) // 128) * 128
    ftt = jnp.zeros((fp, n_pad), jnp.float32).at[:, :n].set(fd_pad.T)
    rb = 200 if n % 200 == 0 else _rows_block(n)
    idx = pl.pallas_call(
        functools.partial(_topk_kernel, n_valid=n, k_sel=k_sel),
        grid=(n // rb,),
        in_specs=[
            pl.BlockSpec((rb, fp), lambda i: (i, 0)),
            pl.BlockSpec((fp, n_pad), lambda i: (0, 0)),
        ],
        out_specs=pl.BlockSpec((rb, 128), lambda i: (i, 0)),
        out_shape=jax.ShapeDtypeStruct((n, 128), jnp.int32),
    )(fd_pad, ftt)
    return idx[:, 1:k_sel]


def _mm_kernel(*refs, n_in, act):
    x = refs[0][...]
    for r in refs[1:n_in]:
        x = x + r[...]
    if act:
        x = _elu(x)
    w = refs[n_in][...]
    refs[n_in + 1][...] = jnp.dot(x, w, preferred_element_type=jnp.float32)


def _run_mm(xs, w, act):
    n, fin = xs[0].shape
    fout = w.shape[1]
    rb = _rows_block(n)
    specs = [pl.BlockSpec((rb, fin), lambda i: (i, 0)) for _ in xs]
    specs.append(pl.BlockSpec((fin, fout), lambda i: (0, 0)))
    return pl.pallas_call(
        functools.partial(_mm_kernel, n_in=len(xs), act=act),
        grid=(n // rb,),
        in_specs=specs,
        out_specs=pl.BlockSpec((rb, fout), lambda i: (i, 0)),
        out_shape=jax.ShapeDtypeStruct((n, fout), jnp.float32),
    )(*xs, w)


def _tail_kernel(h3_ref, hk3_ref, hd3_ref, vw, vb, ow, ob, lng, lnb,
                 pw1, pb1, pg, pbta, pw2, pb2, emb_ref, pred_ref):
    emb = jnp.concatenate(
        [_elu(h3_ref[...]), _elu(hk3_ref[...]), _elu(hd3_ref[...])], axis=1)
    v = jnp.dot(emb, vw[...], preferred_element_type=jnp.float32) + vb[...]
    ao = jnp.dot(v, ow[...], preferred_element_type=jnp.float32) + ob[...]
    hres = emb + ao
    mu = jnp.mean(hres, axis=1, keepdims=True)
    dh = hres - mu
    var = jnp.mean(dh * dh, axis=1, keepdims=True)
    emb2 = dh * jax.lax.rsqrt(var + 1e-5) * lng[...] + lnb[...]
    emb_ref[...] = emb2
    h = _elu((jnp.dot(emb2, pw1[...], preferred_element_type=jnp.float32)
              + pb1[...]) * _BN_INV * pg[...] + pbta[...])
    logits = jnp.dot(h, pw2[...], preferred_element_type=jnp.float32) + pb2[...]
    pred_ref[...] = 1.0 / (1.0 + jnp.exp(-logits))


def _run_tail(p, h3, hk3, hd3):
    n, hdim = h3.shape
    d = 3 * hdim
    drugs = p['pW1'].shape[1]
    dp = ((drugs + 127) // 128) * 128
    pw1 = jnp.zeros((d, dp), jnp.float32).at[:, :drugs].set(p['pW1'])
    pb1 = jnp.zeros((1, dp), jnp.float32).at[:, :drugs].set(p['pb1'][None, :])
    pg = jnp.zeros((1, dp), jnp.float32).at[:, :drugs].set(p['pbn_g'][None, :])
    pbta = jnp.zeros((1, dp), jnp.float32).at[:, :drugs].set(p['pbn_b'][None, :])
    pw2 = jnp.zeros((dp, dp), jnp.float32).at[:drugs, :drugs].set(p['pW2'])
    pb2 = jnp.zeros((1, dp), jnp.float32).at[:, :drugs].set(p['pb2'][None, :])
    rb = 400 if n % 400 == 0 else _rows_block(n)

    def full(a, b):
        return pl.BlockSpec((a, b), lambda i: (0, 0))

    emb, pred = pl.pallas_call(
        _tail_kernel,
        grid=(n // rb,),
        in_specs=[
            pl.BlockSpec((rb, hdim), lambda i: (i, 0)),
            pl.BlockSpec((rb, hdim), lambda i: (i, 0)),
            pl.BlockSpec((rb, hdim), lambda i: (i, 0)),
            full(d, d), full(1, d), full(d, d), full(1, d),
            full(1, d), full(1, d),
            full(d, dp), full(1, dp), full(1, dp), full(1, dp),
            full(dp, dp), full(1, dp),
        ],
        out_specs=[
            pl.BlockSpec((rb, d), lambda i: (i, 0)),
            pl.BlockSpec((rb, dp), lambda i: (i, 0)),
        ],
        out_shape=[
            jax.ShapeDtypeStruct((n, d), jnp.float32),
            jax.ShapeDtypeStruct((n, dp), jnp.float32),
        ],
    )(h3, hk3, hd3,
      p['vW'], p['vb'][None, :], p['oW'], p['ob'][None, :],
      p['ln_g'][None, :], p['ln_b'][None, :],
      pw1, pb1, pg, pbta, pw2, pb2)
    return emb, pred[:, :drugs]


def _agg(x, src, dst, w, n):
    return jax.ops.segment_sum(w[:, None] * x[dst], src, num_segments=n)


def kernel(params, x, adj_indices, adj_values, adj_diff_indices, adj_diff_values):
    p = params
    n = x.shape[0]
    fdim = p['enc_W1'].shape[1] + p['enc_W2'].shape[1] + p['enc_W3'].shape[1]
    fdim_pad = ((fdim + 127) // 128) * 128

    fd_pad, fg1, mse = _run_encoder(p, x, fdim_pad)

    nbr = _run_topk(fd_pad, n, K_NN + 1)

    src = jnp.repeat(jnp.arange(n, dtype=jnp.int32), K_NN)
    dst = nbr.reshape(-1)
    dup = jnp.any(jnp.take(nbr, dst, axis=0) == src[:, None], axis=1)
    w_half = jnp.where(dup, 0.5, 1.0).astype(jnp.float32)
    srcs = jnp.concatenate([src, dst])
    dsts = jnp.concatenate([dst, src])
    ws = jnp.concatenate([w_half, w_half])
    deg = jax.ops.segment_sum(ws, dsts, num_segments=n)
    dinv = jnp.where(deg != 0, deg ** -0.5, 0.0).astype(jnp.float32)
    wknn = ws * dinv[srcs] * dinv[dsts]

    arow, acol = adj_indices[0], adj_indices[1]
    drow, dcol = adj_diff_indices[0], adj_diff_indices[1]

    s1 = _agg(fg1, arow, acol, adj_values, n)
    k1 = _agg(fg1, srcs, dsts, wknn, n)
    d1 = _agg(fg1, drow, dcol, adj_diff_values, n)

    g2w, g3w = p['g2'], p['g3']
    s2 = _agg(_run_mm([s1], g2w, act=True), arow, acol, adj_values, n)
    k2 = _agg(_run_mm([k1], g2w, act=True), srcs, dsts, wknn, n)
    d2 = _agg(_run_mm([d1], g2w, act=True), drow, dcol, adj_diff_values, n)

    s3 = _agg(_run_mm([s2], g3w, act=True), arow, acol, adj_values, n)
    k3 = _agg(_run_mm([k2], g3w, act=True), srcs, dsts, wknn, n)
    d3 = _agg(_run_mm([d2], g3w, act=True), drow, dcol, adj_diff_values, n)

    emb, pred = _run_tail(p, s3, k3, d3)
    return pred, mse, emb
